# baseline (speedup 1.0000x reference)
import functools
import jax
import jax.numpy as jnp
from jax import lax
from jax.experimental import pallas as pl
from jax.experimental.pallas import tpu as pltpu
from jax.experimental.pallas import tpu_sc as plsc

B, L = 4096, 200
E = 64
FAN_H = 16
NUM_HOURS = 25
HN = 32
N_TOK = B * L
P = 1000000 + 1
NW = 32
NS = 4
N_SLAB = N_TOK // NS
PER_W = N_SLAB // NW
CHUNK = 256
N_CHUNKS = PER_W // CHUNK
NBUF = 2
T = 2048
G = N_TOK // T
GS = N_SLAB // T


@functools.lru_cache(maxsize=None)
def _make_sc_gather(slab):
    mesh = plsc.VectorSubcoreMesh(core_axis_name="c", subcore_axis_name="s")
    slab_base = slab * N_SLAB

    @functools.partial(
        pl.kernel,
        mesh=mesh,
        out_type=jax.ShapeDtypeStruct((N_SLAB, E), jnp.float32),
        scratch_types=[
            pltpu.VMEM((NBUF, CHUNK), jnp.int32),
            pltpu.VMEM((NBUF, CHUNK, E), jnp.float32),
            pltpu.SemaphoreType.DMA,
        ],
        compiler_params=pltpu.CompilerParams(use_tc_tiling_on_sc=False),
    )
    def _sc_gather(idx_hbm, table_hbm, out_hbm, idx_v, rows_v, sem):
        wid = lax.axis_index("s") * 2 + lax.axis_index("c")
        base = wid * PER_W

        def body(i, carry):
            slot = lax.rem(i, NBUF)
            off = pl.multiple_of(base + i * CHUNK, CHUNK)
            src = pl.multiple_of(slab_base + off, CHUNK)
            pltpu.sync_copy(idx_hbm.at[pl.ds(src, CHUNK)], idx_v.at[slot])
            pltpu.async_copy(table_hbm.at[idx_v.at[slot]], rows_v.at[slot],
                             sem).wait()
            pltpu.sync_copy(rows_v.at[slot], out_hbm.at[pl.ds(off, CHUNK)])
            return carry

        lax.fori_loop(0, N_CHUNKS, body, 0)

    return _sc_gather


def _dense_body(he_ref, ho_ref, x2_ref, fcw_ref, hpad_ref, b_ref, out_ref):
    x2 = x2_ref[...]
    w1 = fcw_ref[:, :E]
    acce = lax.dot_general(x2[:, :E], w1, (((1,), (1,)), ((), ())),
                           preferred_element_type=jnp.float32)
    acco = lax.dot_general(x2[:, E:], w1, (((1,), (1,)), ((), ())),
                           preferred_element_type=jnp.float32)
    w2 = fcw_ref[:, E:]
    hw = lax.dot_general(hpad_ref[...], w2, (((1,), (1,)), ((), ())),
                         preferred_element_type=jnp.float32)
    hw = hw + b_ref[...]
    ioh = lax.broadcasted_iota(jnp.int32, (HN, T // 2), 0)
    ohe = (ioh == he_ref[0]).astype(jnp.float32)
    oho = (ioh == ho_ref[0]).astype(jnp.float32)
    hce = lax.dot_general(ohe, hw, (((0,), (0,)), ((), ())),
                          preferred_element_type=jnp.float32)
    hco = lax.dot_general(oho, hw, (((0,), (0,)), ((), ())),
                          preferred_element_type=jnp.float32)
    ye = jnp.tanh(acce + hce)
    yo = jnp.tanh(acco + hco)
    out_ref[...] = jnp.concatenate([ye, yo], axis=1)


def _dense_slab(k, out_prev, he3, ho3, x2, fc_w, hour_pad, fc_b2):
    def body(prev_ref, he_ref, ho_ref, x2_ref, fcw_ref, hpad_ref, b_ref,
             out_ref):
        del prev_ref
        _dense_body(he_ref, ho_ref, x2_ref, fcw_ref, hpad_ref, b_ref, out_ref)

    return pl.pallas_call(
        body,
        grid=(GS,),
        in_specs=[
            pl.BlockSpec(memory_space=pl.ANY),
            pl.BlockSpec((1, 1, T // 2), lambda i: (i, 0, 0)),
            pl.BlockSpec((1, 1, T // 2), lambda i: (i, 0, 0)),
            pl.BlockSpec((T // 2, 2 * E), lambda i: (i, 0)),
            pl.BlockSpec((E, E + FAN_H), lambda i: (0, 0)),
            pl.BlockSpec((HN, FAN_H), lambda i: (0, 0)),
            pl.BlockSpec((1, E), lambda i: (0, 0)),
        ],
        out_specs=pl.BlockSpec((T // 2, 2 * E), lambda i: (k * GS + i, 0)),
        out_shape=jax.ShapeDtypeStruct((N_TOK // 2, 2 * E), jnp.float32),
        input_output_aliases={0: 0},
    )(out_prev, he3, ho3, x2, fc_w, hour_pad, fc_b2)


def _dense_first(he3, ho3, x2, fc_w, hour_pad, fc_b2):
    return pl.pallas_call(
        _dense_body,
        grid=(GS,),
        in_specs=[
            pl.BlockSpec((1, 1, T // 2), lambda i: (i, 0, 0)),
            pl.BlockSpec((1, 1, T // 2), lambda i: (i, 0, 0)),
            pl.BlockSpec((T // 2, 2 * E), lambda i: (i, 0)),
            pl.BlockSpec((E, E + FAN_H), lambda i: (0, 0)),
            pl.BlockSpec((HN, FAN_H), lambda i: (0, 0)),
            pl.BlockSpec((1, E), lambda i: (0, 0)),
        ],
        out_specs=pl.BlockSpec((T // 2, 2 * E), lambda i: (i, 0)),
        out_shape=jax.ShapeDtypeStruct((N_TOK // 2, 2 * E), jnp.float32),
    )(he3, ho3, x2, fc_w, hour_pad, fc_b2)


def kernel(token_seq, hour_seq, poi_table, hour_table, fc_w, fc_b):
    tok = jnp.asarray(token_seq, jnp.int32).reshape(N_TOK)
    hour = jnp.asarray(hour_seq, jnp.int32).reshape(N_TOK)
    he = hour[0::2]
    ho = hour[1::2]
    hour_pad = jnp.pad(hour_table.astype(jnp.float32),
                       ((0, HN - NUM_HOURS), (0, 0)))
    fc_b2 = fc_b.astype(jnp.float32).reshape(1, E)
    fcw = fc_w.astype(jnp.float32)
    tbl = poi_table.astype(jnp.float32)

    x2s, he3s, ho3s = [], [], []
    he4 = he.reshape(NS, GS, 1, T // 2)
    ho4 = ho.reshape(NS, GS, 1, T // 2)
    for k in range(NS):
        xk = _make_sc_gather(k)(tok, tbl)
        x2s.append(xk.reshape(N_SLAB // 2, 2 * E))
        he3s.append(he4[k])
        ho3s.append(ho4[k])

    out = _dense_first(he3s[0], ho3s[0], x2s[0], fcw, hour_pad, fc_b2)
    for k in range(1, NS):
        out = _dense_slab(k, out, he3s[k], ho3s[k], x2s[k], fcw, hour_pad,
                          fc_b2)
    return out.reshape(B, L, E)


# speedup vs baseline: 1.0870x; 1.0870x over previous
import functools
import jax
import jax.numpy as jnp
from jax import lax
from jax.experimental import pallas as pl
from jax.experimental.pallas import tpu as pltpu
from jax.experimental.pallas import tpu_sc as plsc

B, L = 4096, 200
E = 64
FAN_H = 16
NUM_HOURS = 25
HN = 32
N_TOK = B * L
P = 1000000 + 1
NW = 32
NS = 4
N_SLAB = N_TOK // NS
PER_W = N_SLAB // NW
CHUNK = 256
N_CHUNKS = PER_W // CHUNK
NBUF = 2
T = 2048
G = N_TOK // T
GS = N_SLAB // T


@functools.lru_cache(maxsize=None)
def _make_sc_gather(slab):
    mesh = plsc.VectorSubcoreMesh(core_axis_name="c", subcore_axis_name="s")
    slab_base = slab * N_SLAB

    @functools.partial(
        pl.kernel,
        mesh=mesh,
        out_type=jax.ShapeDtypeStruct((N_SLAB, E), jnp.float32),
        scratch_types=[
            pltpu.VMEM((NBUF, CHUNK), jnp.int32),
            pltpu.VMEM((NBUF, CHUNK, E), jnp.float32),
            pltpu.SemaphoreType.DMA,
        ],
        compiler_params=pltpu.CompilerParams(use_tc_tiling_on_sc=False),
    )
    def _sc_gather(idx_hbm, table_hbm, out_hbm, idx_v, rows_v, sem):
        wid = lax.axis_index("s") * 2 + lax.axis_index("c")
        base = wid * PER_W

        def body(i, carry):
            slot = lax.rem(i, NBUF)
            off = pl.multiple_of(base + i * CHUNK, CHUNK)
            src = pl.multiple_of(slab_base + off, CHUNK)
            pltpu.sync_copy(idx_hbm.at[pl.ds(src, CHUNK)], idx_v.at[slot])
            pltpu.async_copy(table_hbm.at[idx_v.at[slot]], rows_v.at[slot],
                             sem).wait()
            pltpu.sync_copy(rows_v.at[slot], out_hbm.at[pl.ds(off, CHUNK)])
            return carry

        lax.fori_loop(0, N_CHUNKS, body, 0)

    return _sc_gather


def _dense_body(he_ref, ho_ref, x2_ref, fcw_ref, hpad_ref, b_ref, out_ref):
    x2 = x2_ref[...]
    w1 = fcw_ref[:, :E]
    acce = lax.dot_general(x2[:, :E], w1, (((1,), (1,)), ((), ())),
                           preferred_element_type=jnp.float32)
    acco = lax.dot_general(x2[:, E:], w1, (((1,), (1,)), ((), ())),
                           preferred_element_type=jnp.float32)
    w2 = fcw_ref[:, E:]
    hw = lax.dot_general(hpad_ref[...], w2, (((1,), (1,)), ((), ())),
                         preferred_element_type=jnp.float32)
    hw = hw + b_ref[...]
    ioh = lax.broadcasted_iota(jnp.int32, (HN, T // 2), 0)
    ohe = (ioh == he_ref[0]).astype(jnp.float32)
    oho = (ioh == ho_ref[0]).astype(jnp.float32)
    hce = lax.dot_general(ohe, hw, (((0,), (0,)), ((), ())),
                          preferred_element_type=jnp.float32)
    hco = lax.dot_general(oho, hw, (((0,), (0,)), ((), ())),
                          preferred_element_type=jnp.float32)
    ye = jnp.tanh(acce + hce)
    yo = jnp.tanh(acco + hco)
    out_ref[...] = jnp.stack([ye, yo], axis=1).reshape(T, E)


def _dense_slab(k, out_prev, he3, ho3, x2, fc_w, hour_pad, fc_b2):
    def body(prev_ref, he_ref, ho_ref, x2_ref, fcw_ref, hpad_ref, b_ref,
             out_ref):
        del prev_ref
        _dense_body(he_ref, ho_ref, x2_ref, fcw_ref, hpad_ref, b_ref, out_ref)

    return pl.pallas_call(
        body,
        grid=(GS,),
        in_specs=[
            pl.BlockSpec(memory_space=pl.ANY),
            pl.BlockSpec((1, 1, T // 2), lambda i: (i, 0, 0)),
            pl.BlockSpec((1, 1, T // 2), lambda i: (i, 0, 0)),
            pl.BlockSpec((T // 2, 2 * E), lambda i: (i, 0)),
            pl.BlockSpec((E, E + FAN_H), lambda i: (0, 0)),
            pl.BlockSpec((HN, FAN_H), lambda i: (0, 0)),
            pl.BlockSpec((1, E), lambda i: (0, 0)),
        ],
        out_specs=pl.BlockSpec((T, E), lambda i: (k * GS + i, 0)),
        out_shape=jax.ShapeDtypeStruct((N_TOK, E), jnp.float32),
        input_output_aliases={0: 0},
    )(out_prev, he3, ho3, x2, fc_w, hour_pad, fc_b2)


def _dense_first(he3, ho3, x2, fc_w, hour_pad, fc_b2):
    return pl.pallas_call(
        _dense_body,
        grid=(GS,),
        in_specs=[
            pl.BlockSpec((1, 1, T // 2), lambda i: (i, 0, 0)),
            pl.BlockSpec((1, 1, T // 2), lambda i: (i, 0, 0)),
            pl.BlockSpec((T // 2, 2 * E), lambda i: (i, 0)),
            pl.BlockSpec((E, E + FAN_H), lambda i: (0, 0)),
            pl.BlockSpec((HN, FAN_H), lambda i: (0, 0)),
            pl.BlockSpec((1, E), lambda i: (0, 0)),
        ],
        out_specs=pl.BlockSpec((T, E), lambda i: (i, 0)),
        out_shape=jax.ShapeDtypeStruct((N_TOK, E), jnp.float32),
    )(he3, ho3, x2, fc_w, hour_pad, fc_b2)


def kernel(token_seq, hour_seq, poi_table, hour_table, fc_w, fc_b):
    tok = jnp.asarray(token_seq, jnp.int32).reshape(N_TOK)
    hour = jnp.asarray(hour_seq, jnp.int32).reshape(N_TOK)
    he = hour[0::2]
    ho = hour[1::2]
    hour_pad = jnp.pad(hour_table.astype(jnp.float32),
                       ((0, HN - NUM_HOURS), (0, 0)))
    fc_b2 = fc_b.astype(jnp.float32).reshape(1, E)
    fcw = fc_w.astype(jnp.float32)
    tbl = poi_table.astype(jnp.float32)

    x2s, he3s, ho3s = [], [], []
    he4 = he.reshape(NS, GS, 1, T // 2)
    ho4 = ho.reshape(NS, GS, 1, T // 2)
    for k in range(NS):
        xk = _make_sc_gather(k)(tok, tbl)
        x2s.append(xk.reshape(N_SLAB // 2, 2 * E))
        he3s.append(he4[k])
        ho3s.append(ho4[k])

    out = _dense_first(he3s[0], ho3s[0], x2s[0], fcw, hour_pad, fc_b2)
    for k in range(1, NS):
        out = _dense_slab(k, out, he3s[k], ho3s[k], x2s[k], fcw, hour_pad,
                          fc_b2)
    return out.reshape(B, L, E)


# speedup vs baseline: 1.1460x; 1.0543x over previous
import functools
import jax
import jax.numpy as jnp
from jax import lax
from jax.experimental import pallas as pl
from jax.experimental.pallas import tpu as pltpu
from jax.experimental.pallas import tpu_sc as plsc

B, L = 4096, 200
E = 64
FAN_H = 16
NUM_HOURS = 25
HN = 32
N_TOK = B * L
P = 1000000 + 1
NW = 32
NS = 8
N_SLAB = N_TOK // NS
PER_W = N_SLAB // NW
CHUNK = 320
N_CHUNKS = PER_W // CHUNK
NBUF = 2
T = 4096
G = N_TOK // T
GS = N_SLAB // T


@functools.lru_cache(maxsize=None)
def _make_sc_gather(slab):
    mesh = plsc.VectorSubcoreMesh(core_axis_name="c", subcore_axis_name="s")
    slab_base = slab * N_SLAB

    @functools.partial(
        pl.kernel,
        mesh=mesh,
        out_type=jax.ShapeDtypeStruct((N_SLAB, E), jnp.float32),
        scratch_types=[
            pltpu.VMEM((NBUF, CHUNK), jnp.int32),
            pltpu.VMEM((NBUF, CHUNK, E), jnp.float32),
            pltpu.SemaphoreType.DMA,
        ],
        compiler_params=pltpu.CompilerParams(use_tc_tiling_on_sc=False),
    )
    def _sc_gather(idx_hbm, table_hbm, out_hbm, idx_v, rows_v, sem):
        wid = lax.axis_index("s") * 2 + lax.axis_index("c")
        base = wid * PER_W

        def body(i, carry):
            slot = lax.rem(i, NBUF)
            off = pl.multiple_of(base + i * CHUNK, CHUNK)
            src = pl.multiple_of(slab_base + off, CHUNK)
            pltpu.sync_copy(idx_hbm.at[pl.ds(src, CHUNK)], idx_v.at[slot])
            pltpu.async_copy(table_hbm.at[idx_v.at[slot]], rows_v.at[slot],
                             sem).wait()
            pltpu.sync_copy(rows_v.at[slot], out_hbm.at[pl.ds(off, CHUNK)])
            return carry

        lax.fori_loop(0, N_CHUNKS, body, 0)

    return _sc_gather


def _dense_body(he_ref, ho_ref, x2_ref, fcw_ref, hpad_ref, b_ref, out_ref):
    x2 = x2_ref[...]
    w1 = fcw_ref[:, :E]
    acce = lax.dot_general(x2[:, :E], w1, (((1,), (1,)), ((), ())),
                           preferred_element_type=jnp.float32)
    acco = lax.dot_general(x2[:, E:], w1, (((1,), (1,)), ((), ())),
                           preferred_element_type=jnp.float32)
    w2 = fcw_ref[:, E:]
    hw = lax.dot_general(hpad_ref[...], w2, (((1,), (1,)), ((), ())),
                         preferred_element_type=jnp.float32)
    hw = hw + b_ref[...]
    ioh = lax.broadcasted_iota(jnp.int32, (HN, T // 2), 0)
    ohe = (ioh == he_ref[0]).astype(jnp.float32)
    oho = (ioh == ho_ref[0]).astype(jnp.float32)
    hce = lax.dot_general(ohe, hw, (((0,), (0,)), ((), ())),
                          preferred_element_type=jnp.float32)
    hco = lax.dot_general(oho, hw, (((0,), (0,)), ((), ())),
                          preferred_element_type=jnp.float32)
    ye = jnp.tanh(acce + hce)
    yo = jnp.tanh(acco + hco)
    out_ref[...] = jnp.stack([ye, yo], axis=1).reshape(T, E)


def _dense_slab(k, out_prev, he3, ho3, x2, fc_w, hour_pad, fc_b2):
    def body(prev_ref, he_ref, ho_ref, x2_ref, fcw_ref, hpad_ref, b_ref,
             out_ref):
        del prev_ref
        _dense_body(he_ref, ho_ref, x2_ref, fcw_ref, hpad_ref, b_ref, out_ref)

    return pl.pallas_call(
        body,
        grid=(GS,),
        in_specs=[
            pl.BlockSpec(memory_space=pl.ANY),
            pl.BlockSpec((1, 1, T // 2), lambda i: (i, 0, 0)),
            pl.BlockSpec((1, 1, T // 2), lambda i: (i, 0, 0)),
            pl.BlockSpec((T // 2, 2 * E), lambda i: (i, 0)),
            pl.BlockSpec((E, E + FAN_H), lambda i: (0, 0)),
            pl.BlockSpec((HN, FAN_H), lambda i: (0, 0)),
            pl.BlockSpec((1, E), lambda i: (0, 0)),
        ],
        out_specs=pl.BlockSpec((T, E), lambda i: (k * GS + i, 0)),
        out_shape=jax.ShapeDtypeStruct((N_TOK, E), jnp.float32),
        input_output_aliases={0: 0},
    )(out_prev, he3, ho3, x2, fc_w, hour_pad, fc_b2)


def _dense_first(he3, ho3, x2, fc_w, hour_pad, fc_b2):
    return pl.pallas_call(
        _dense_body,
        grid=(GS,),
        in_specs=[
            pl.BlockSpec((1, 1, T // 2), lambda i: (i, 0, 0)),
            pl.BlockSpec((1, 1, T // 2), lambda i: (i, 0, 0)),
            pl.BlockSpec((T // 2, 2 * E), lambda i: (i, 0)),
            pl.BlockSpec((E, E + FAN_H), lambda i: (0, 0)),
            pl.BlockSpec((HN, FAN_H), lambda i: (0, 0)),
            pl.BlockSpec((1, E), lambda i: (0, 0)),
        ],
        out_specs=pl.BlockSpec((T, E), lambda i: (i, 0)),
        out_shape=jax.ShapeDtypeStruct((N_TOK, E), jnp.float32),
    )(he3, ho3, x2, fc_w, hour_pad, fc_b2)


def kernel(token_seq, hour_seq, poi_table, hour_table, fc_w, fc_b):
    tok = jnp.asarray(token_seq, jnp.int32).reshape(N_TOK)
    hour = jnp.asarray(hour_seq, jnp.int32).reshape(N_TOK)
    he = hour[0::2]
    ho = hour[1::2]
    hour_pad = jnp.pad(hour_table.astype(jnp.float32),
                       ((0, HN - NUM_HOURS), (0, 0)))
    fc_b2 = fc_b.astype(jnp.float32).reshape(1, E)
    fcw = fc_w.astype(jnp.float32)
    tbl = poi_table.astype(jnp.float32)

    x2s, he3s, ho3s = [], [], []
    he4 = he.reshape(NS, GS, 1, T // 2)
    ho4 = ho.reshape(NS, GS, 1, T // 2)
    for k in range(NS):
        xk = _make_sc_gather(k)(tok, tbl)
        x2s.append(xk.reshape(N_SLAB // 2, 2 * E))
        he3s.append(he4[k])
        ho3s.append(ho4[k])

    out = _dense_first(he3s[0], ho3s[0], x2s[0], fcw, hour_pad, fc_b2)
    for k in range(1, NS):
        out = _dense_slab(k, out, he3s[k], ho3s[k], x2s[k], fcw, hour_pad,
                          fc_b2)
    return out.reshape(B, L, E)
